# baseline (speedup 1.0000x reference)
import functools
import math

import jax
import jax.numpy as jnp
from jax import lax
from jax.experimental import pallas as pl
from jax.experimental.pallas import tpu as pltpu
from jax.experimental.pallas import tpu_sc as plsc

EMB = 64
HALF = 32
BATCH = 16384
MAX_PERIOD = 10000.0

_NC = 2
_NS = 16
_NW = _NC * _NS
_BPW = BATCH // _NW
_CHUNK = 128
_NCH = _BPW // _CHUNK

@functools.cache
def _make_sc_gather():
    mesh = plsc.VectorSubcoreMesh(core_axis_name="c", subcore_axis_name="s")

    @functools.partial(
        pl.kernel,
        mesh=mesh,
        out_type=jax.ShapeDtypeStruct((BATCH, EMB), jnp.float32),
        scratch_types=[
            pltpu.VMEM((_NCH, _CHUNK), jnp.int32),
            pltpu.VMEM((_BPW, EMB), jnp.float32),
            pltpu.SemaphoreType.DMA,
        ],
        compiler_params=pltpu.CompilerParams(use_tc_tiling_on_sc=False),
    )
    def _sc_gather(labels_hbm, table_hbm, out_hbm, idx_v, rows_v, sem):
        wid = lax.axis_index("s") * _NC + lax.axis_index("c")
        pltpu.sync_copy(labels_hbm.at[pl.ds(wid * _NCH, _NCH)], idx_v)
        copies = []
        for j in range(_NCH):
            copies.append(
                pltpu.async_copy(
                    table_hbm.at[idx_v.at[j]],
                    rows_v.at[pl.ds(j * _CHUNK, _CHUNK)],
                    sem,
                )
            )
        for c in copies:
            c.wait()
        pltpu.sync_copy(rows_v, out_hbm.at[pl.ds(wid * _BPW, _BPW)])

    return _sc_gather


_BLK = 1024


def _tc_body(t_ref, g_ref, o_ref):
    t = t_ref[:, :].astype(jnp.float32)
    j = lax.broadcasted_iota(jnp.int32, (_BLK, EMB), 1)
    k = jnp.where(j < HALF, j, j - HALF).astype(jnp.float32)
    freqs = jnp.exp(k * (-math.log(MAX_PERIOD) / HALF))
    args = t * freqs
    emb = jnp.where(j < HALF, jnp.cos(args), jnp.sin(args))
    o_ref[:, :] = emb + g_ref[:, :]


_tc_add = pl.pallas_call(
    _tc_body,
    grid=(BATCH // _BLK,),
    in_specs=[
        pl.BlockSpec((_BLK, 1), lambda i: (i, 0)),
        pl.BlockSpec((_BLK, EMB), lambda i: (i, 0)),
    ],
    out_specs=pl.BlockSpec((_BLK, EMB), lambda i: (i, 0)),
    out_shape=jax.ShapeDtypeStruct((BATCH, EMB), jnp.float32),
)


def kernel(timesteps, labels, class_embedding):
    labels2d = labels.reshape(_NW * _NCH, _CHUNK)
    gathered = _make_sc_gather()(labels2d, class_embedding)
    t2d = timesteps.reshape(BATCH, 1)
    return _tc_add(t2d, gathered)


# speedup vs baseline: 1.5977x; 1.5977x over previous
import functools
import math

import jax
import jax.numpy as jnp
from jax import lax
from jax.experimental import pallas as pl
from jax.experimental.pallas import tpu as pltpu
from jax.experimental.pallas import tpu_sc as plsc

EMB = 64
HALF = 32
BATCH = 16384
MAX_PERIOD = 10000.0

_NC = 2
_NS = 16
_NW = _NC * _NS
_BPW = BATCH // _NW
_FIRE = 16
_NBATCH = _BPW // _FIRE


@functools.cache
def _make_sc_gather():
    mesh = plsc.VectorSubcoreMesh(core_axis_name="c", subcore_axis_name="s")

    @functools.partial(
        pl.kernel,
        mesh=mesh,
        out_type=jax.ShapeDtypeStruct((BATCH, EMB), jnp.float32),
        scratch_types=[
            pltpu.VMEM((_BPW,), jnp.int32),
            pltpu.VMEM((_BPW, EMB), jnp.float32),
            pltpu.SemaphoreType.DMA,
        ],
    )
    def _sc_gather(labels_hbm, table_hbm, out_hbm, labels_v, rows_v, sem):
        wid = lax.axis_index("s") * _NC + lax.axis_index("c")
        base = wid * _BPW
        pltpu.sync_copy(labels_hbm.at[pl.ds(base, _BPW)], labels_v)

        def batch(c, carry):
            i0 = c * _FIRE
            lvec = labels_v[pl.ds(i0, _FIRE)]
            copies = []
            for k in range(_FIRE):
                lbl = lvec[k]
                copies.append(
                    pltpu.async_copy(
                        table_hbm.at[pl.ds(lbl, 1)],
                        rows_v.at[pl.ds(i0 + k, 1)],
                        sem,
                    )
                )
            for cp in copies:
                cp.wait()
            return carry

        lax.fori_loop(0, _NBATCH, batch, 0)
        pltpu.sync_copy(rows_v, out_hbm.at[pl.ds(base, _BPW)])

    return _sc_gather


_BLK = 1024


def _tc_body(t_ref, g_ref, o_ref):
    t = t_ref[:, :].astype(jnp.float32)
    j = lax.broadcasted_iota(jnp.int32, (_BLK, EMB), 1)
    k = jnp.where(j < HALF, j, j - HALF).astype(jnp.float32)
    freqs = jnp.exp(k * (-math.log(MAX_PERIOD) / HALF))
    args = t * freqs
    emb = jnp.where(j < HALF, jnp.cos(args), jnp.sin(args))
    o_ref[:, :] = emb + g_ref[:, :]


_tc_add = pl.pallas_call(
    _tc_body,
    grid=(BATCH // _BLK,),
    in_specs=[
        pl.BlockSpec((_BLK, 1), lambda i: (i, 0)),
        pl.BlockSpec((_BLK, EMB), lambda i: (i, 0)),
    ],
    out_specs=pl.BlockSpec((_BLK, EMB), lambda i: (i, 0)),
    out_shape=jax.ShapeDtypeStruct((BATCH, EMB), jnp.float32),
)


def kernel(timesteps, labels, class_embedding):
    gathered = _make_sc_gather()(labels, class_embedding)
    t2d = timesteps.reshape(BATCH, 1)
    return _tc_add(t2d, gathered)


# speedup vs baseline: 2.8531x; 1.7858x over previous
import functools
import math

import jax
import jax.numpy as jnp
from jax import lax
from jax.experimental import pallas as pl
from jax.experimental.pallas import tpu as pltpu
from jax.experimental.pallas import tpu_sc as plsc

EMB = 64
HALF = 32
BATCH = 16384
MAX_PERIOD = 10000.0

_NC = 2
_NS = 16
_NW = _NC * _NS
_BPW = BATCH // _NW
_RING = 8
_SEG = 128
_NSEG = _BPW // _SEG
_NGRP = _SEG // _RING


@functools.cache
def _make_sc_gather():
    mesh = plsc.VectorSubcoreMesh(core_axis_name="c", subcore_axis_name="s")

    slab_types = [pltpu.VMEM((EMB, 128), jnp.float32) for _ in range(_RING)]
    gsem_types = [pltpu.SemaphoreType.DMA for _ in range(_RING)]

    @functools.partial(
        pl.kernel,
        mesh=mesh,
        out_type=jax.ShapeDtypeStruct((EMB, BATCH), jnp.float32),
        scratch_types=[
            pltpu.VMEM((_BPW + 8, ), jnp.int32),
            pltpu.VMEM((EMB, 128), jnp.float32),
            pltpu.VMEM((EMB, 128), jnp.float32),
            *slab_types,
            *gsem_types,
            pltpu.SemaphoreType.DMA,
            pltpu.SemaphoreType.DMA,
        ],
        compiler_params=pltpu.CompilerParams(needs_layout_passes=False),
    )
    def _sc_gather(labels_hbm, tablet_hbm, out_hbm,
                   labels_v, col_a, col_b, *rest):
        slabs = rest[:_RING]
        gsems = rest[_RING:2 * _RING]
        osems = rest[2 * _RING:]
        cols = (col_a, col_b)
        wid = lax.axis_index("s") * _NC + lax.axis_index("c")
        base = wid * _BPW
        pltpu.sync_copy(labels_hbm.at[pl.ds(base, _BPW)],
                        labels_v.at[pl.ds(0, _BPW)])

        rows16 = [lax.iota(jnp.int32, 16) + 16 * q for q in range(4)]

        def fire(slot, lbl):
            off = pl.multiple_of(
                lax.shift_right_logical(lbl, 7) * 128, 128)
            pltpu.async_copy(
                tablet_hbm.at[:, pl.ds(off, 128)], slabs[slot], gsems[slot])

        def wait_slot(slot):
            pltpu.make_async_copy(
                tablet_hbm.at[:, pl.ds(0, 128)], slabs[slot],
                gsems[slot]).wait()

        def extract(slot, lbl, colbuf, col):
            lane = jnp.broadcast_to(lbl & 127, (16,))
            colv = jnp.broadcast_to(col, (16,))
            for q in range(4):
                vals = plsc.load_gather(slabs[slot], [rows16[q], lane])
                plsc.store_scatter(colbuf, [rows16[q], colv], vals)

        out_copies = [None, None]
        for s in range(_NSEG):
            cb = s % 2
            if out_copies[cb] is not None:
                out_copies[cb].wait()
            lv0 = labels_v[pl.ds(s * _SEG, 16)]
            for j in range(_RING):
                fire(j, lv0[j])

            def group(c, carry, s=s, cb=cb):
                lv = labels_v[pl.ds(s * _SEG + c * _RING, 16)]
                for j in range(_RING):
                    wait_slot(j)
                    extract(j, lv[j], cols[cb], c * _RING + j)

                    @pl.when(c < _NGRP - 1)
                    def _():
                        fire(j, lv[_RING + j])
                return carry

            lax.fori_loop(0, _NGRP, group, 0)
            out_copies[cb] = pltpu.async_copy(
                cols[cb],
                out_hbm.at[:, pl.ds(base + s * _SEG, _SEG)],
                osems[cb],
            )
        out_copies[0].wait()
        out_copies[1].wait()

    return _sc_gather


_BLK = 512


def _tc_body(t_ref, g_ref, o_ref):
    t = t_ref[0, 0, :].astype(jnp.float32).reshape(1, _BLK)
    tb = jnp.broadcast_to(t, (EMB, _BLK))
    j = lax.broadcasted_iota(jnp.int32, (EMB, _BLK), 0)
    k = jnp.where(j < HALF, j, j - HALF).astype(jnp.float32)
    freqs = jnp.exp(k * (-math.log(MAX_PERIOD) / HALF))
    args = tb * freqs
    emb = jnp.where(j < HALF, jnp.cos(args), jnp.sin(args))
    o_ref[:, :] = emb + g_ref[:, :]


_tc_add = pl.pallas_call(
    _tc_body,
    grid=(BATCH // _BLK,),
    in_specs=[
        pl.BlockSpec((1, 1, _BLK), lambda i: (i, 0, 0)),
        pl.BlockSpec((EMB, _BLK), lambda i: (0, i)),
    ],
    out_specs=pl.BlockSpec((EMB, _BLK), lambda i: (0, i)),
    out_shape=jax.ShapeDtypeStruct((EMB, BATCH), jnp.float32),
)


def kernel(timesteps, labels, class_embedding):
    gathered_t = _make_sc_gather()(labels, class_embedding.T)
    t3d = timesteps.reshape(BATCH // _BLK, 1, _BLK)
    return _tc_add(t3d, gathered_t).T


# speedup vs baseline: 2.8861x; 1.0116x over previous
import functools
import math

import jax
import jax.numpy as jnp
from jax import lax
from jax.experimental import pallas as pl
from jax.experimental.pallas import tpu as pltpu
from jax.experimental.pallas import tpu_sc as plsc

EMB = 64
HALF = 32
BATCH = 16384
MAX_PERIOD = 10000.0

_NC = 2
_NS = 16
_NW = _NC * _NS
_BPW = BATCH // _NW
_RING = 8
_SEG = 128
_NSEG = _BPW // _SEG
_NGRP = _SEG // _RING


@functools.cache
def _make_sc_gather():
    mesh = plsc.VectorSubcoreMesh(core_axis_name="c", subcore_axis_name="s")

    slab_types = [pltpu.VMEM((EMB, 128), jnp.float32) for _ in range(_RING)]
    gsem_types = [pltpu.SemaphoreType.DMA for _ in range(_RING)]

    @functools.partial(
        pl.kernel,
        mesh=mesh,
        out_type=jax.ShapeDtypeStruct((EMB, BATCH), jnp.float32),
        scratch_types=[
            pltpu.VMEM((_BPW + 8, ), jnp.int32),
            pltpu.VMEM((EMB, 128), jnp.float32),
            pltpu.VMEM((EMB, 128), jnp.float32),
            *slab_types,
            *gsem_types,
            pltpu.SemaphoreType.DMA,
            pltpu.SemaphoreType.DMA,
        ],
        compiler_params=pltpu.CompilerParams(needs_layout_passes=False),
    )
    def _sc_gather(labels_hbm, tablet_hbm, out_hbm,
                   labels_v, col_a, col_b, *rest):
        slabs = rest[:_RING]
        gsems = rest[_RING:2 * _RING]
        osems = rest[2 * _RING:]
        cols = (col_a, col_b)
        wid = lax.axis_index("s") * _NC + lax.axis_index("c")
        base = wid * _BPW
        pltpu.sync_copy(labels_hbm.at[pl.ds(base, _BPW)],
                        labels_v.at[pl.ds(0, _BPW)])

        rows16 = [lax.iota(jnp.int32, 16) + 16 * q for q in range(4)]

        def fire(slot, lbl):
            off = pl.multiple_of(
                lax.shift_right_logical(lbl, 7) * 128, 128)
            pltpu.async_copy(
                tablet_hbm.at[:, pl.ds(off, 128)], slabs[slot], gsems[slot])

        def wait_slot(slot):
            pltpu.make_async_copy(
                tablet_hbm.at[:, pl.ds(0, 128)], slabs[slot],
                gsems[slot]).wait()

        def extract(slot, lbl, colbuf, col):
            lane = jnp.broadcast_to(lbl & 127, (16,))
            colv = jnp.broadcast_to(col, (16,))
            for q in range(4):
                vals = plsc.load_gather(slabs[slot], [rows16[q], lane])
                plsc.store_scatter(colbuf, [rows16[q], colv], vals)

        out_copies = [None, None]
        for s in range(_NSEG):
            cb = s % 2
            if out_copies[cb] is not None:
                out_copies[cb].wait()
            lv0 = labels_v[pl.ds(s * _SEG, 16)]
            for j in range(_RING):
                fire(j, lv0[j])

            def group(c, carry, s=s, cb=cb):
                lv = labels_v[pl.ds(s * _SEG + c * _RING, 16)]
                for j in range(_RING):
                    wait_slot(j)
                    extract(j, lv[j], cols[cb], c * _RING + j)

                    @pl.when(c < _NGRP - 1)
                    def _():
                        fire(j, lv[_RING + j])
                return carry

            lax.fori_loop(0, _NGRP, group, 0)
            out_copies[cb] = pltpu.async_copy(
                cols[cb],
                out_hbm.at[:, pl.ds(base + s * _SEG, _SEG)],
                osems[cb],
            )
        out_copies[0].wait()
        out_copies[1].wait()

    return _sc_gather


_BLK = 512


def _tc_body(t_ref, g_ref, o_ref):
    t = t_ref[0, 0, :].astype(jnp.float32).reshape(1, _BLK)
    tb = jnp.broadcast_to(t, (HALF, _BLK))
    k = lax.broadcasted_iota(jnp.int32, (HALF, _BLK), 0).astype(jnp.float32)
    freqs = jnp.exp(k * (-math.log(MAX_PERIOD) / HALF))
    args = tb * freqs
    o_ref[0:HALF, :] = jnp.cos(args) + g_ref[0:HALF, :]
    o_ref[HALF:EMB, :] = jnp.sin(args) + g_ref[HALF:EMB, :]


_tc_add = pl.pallas_call(
    _tc_body,
    grid=(BATCH // _BLK,),
    in_specs=[
        pl.BlockSpec((1, 1, _BLK), lambda i: (i, 0, 0)),
        pl.BlockSpec((EMB, _BLK), lambda i: (0, i)),
    ],
    out_specs=pl.BlockSpec((EMB, _BLK), lambda i: (0, i)),
    out_shape=jax.ShapeDtypeStruct((EMB, BATCH), jnp.float32),
)


def kernel(timesteps, labels, class_embedding):
    gathered_t = _make_sc_gather()(labels, class_embedding.T)
    t3d = timesteps.reshape(BATCH // _BLK, 1, _BLK)
    return _tc_add(t3d, gathered_t).T


# speedup vs baseline: 3.0652x; 1.0621x over previous
import functools
import math

import jax
import jax.numpy as jnp
from jax import lax
from jax.experimental import pallas as pl
from jax.experimental.pallas import tpu as pltpu
from jax.experimental.pallas import tpu_sc as plsc

EMB = 64
HALF = 32
BATCH = 16384
MAX_PERIOD = 10000.0
TMAX = 1024

_NC = 2
_NS = 16
_NW = _NC * _NS
_BPW = BATCH // _NW
_RING = 8
_SEG = 128
_NSEG = _BPW // _SEG
_NGRP = _SEG // _RING


def _emb_body(o_ref):
    r = lax.broadcasted_iota(jnp.int32, (HALF, TMAX), 0)
    t = lax.broadcasted_iota(jnp.int32, (HALF, TMAX), 1).astype(jnp.float32)
    k0 = jnp.where(r < 16, 2 * r, 2 * r - HALF).astype(jnp.float32)
    c = -math.log(MAX_PERIOD) / HALF
    a0 = t * jnp.exp(k0 * c)
    a1 = t * jnp.exp((k0 + 1.0) * c)
    v0 = jnp.where(r < 16, jnp.cos(a0), jnp.sin(a0))
    v1 = jnp.where(r < 16, jnp.cos(a1), jnp.sin(a1))
    u0 = lax.bitcast_convert_type(v0.astype(jnp.bfloat16), jnp.uint16)
    u1 = lax.bitcast_convert_type(v1.astype(jnp.bfloat16), jnp.uint16)
    packed = u0.astype(jnp.int32) | lax.shift_left(u1.astype(jnp.int32), 16)
    o_ref[:, :] = packed


_emb_table = pl.pallas_call(
    _emb_body,
    out_shape=jax.ShapeDtypeStruct((HALF, TMAX), jnp.int32),
)


@functools.cache
def _make_sc_gather():
    mesh = plsc.VectorSubcoreMesh(core_axis_name="c", subcore_axis_name="s")

    slab_types = [pltpu.VMEM((EMB, 128), jnp.float32) for _ in range(_RING)]
    gsem_types = [pltpu.SemaphoreType.DMA for _ in range(_RING)]

    @functools.partial(
        pl.kernel,
        mesh=mesh,
        out_type=jax.ShapeDtypeStruct((EMB, BATCH), jnp.float32),
        scratch_types=[
            pltpu.VMEM((_BPW + 8, ), jnp.int32),
            pltpu.VMEM((_BPW + 8, ), jnp.int32),
            pltpu.VMEM((HALF, TMAX), jnp.int32),
            pltpu.VMEM((EMB, 128), jnp.float32),
            pltpu.VMEM((EMB, 128), jnp.float32),
            *slab_types,
            *gsem_types,
            pltpu.SemaphoreType.DMA,
            pltpu.SemaphoreType.DMA,
        ],
        compiler_params=pltpu.CompilerParams(needs_layout_passes=False),
    )
    def _sc_gather(labels_hbm, times_hbm, embp_hbm, tablet_hbm, out_hbm,
                   labels_v, times_v, emb_v, col_a, col_b, *rest):
        slabs = rest[:_RING]
        gsems = rest[_RING:2 * _RING]
        osems = rest[2 * _RING:]
        cols = (col_a, col_b)
        wid = lax.axis_index("s") * _NC + lax.axis_index("c")
        base = wid * _BPW
        pltpu.sync_copy(labels_hbm.at[pl.ds(base, _BPW)],
                        labels_v.at[pl.ds(0, _BPW)])
        pltpu.sync_copy(times_hbm.at[pl.ds(base, _BPW)],
                        times_v.at[pl.ds(0, _BPW)])
        pltpu.sync_copy(embp_hbm, emb_v)

        rows16 = [lax.iota(jnp.int32, 16) + 16 * q for q in range(4)]
        rhalf = [lax.shift_right_logical(rows16[q], 1) for q in range(4)]
        odd16 = [(rows16[q] & 1) == 1 for q in range(4)]
        himask = jnp.full((16,), -65536, jnp.int32)

        def fire(slot, lbl):
            off = pl.multiple_of(
                lax.shift_right_logical(lbl, 7) * 128, 128)
            pltpu.async_copy(
                tablet_hbm.at[:, pl.ds(off, 128)], slabs[slot], gsems[slot])

        def wait_slot(slot):
            pltpu.make_async_copy(
                tablet_hbm.at[:, pl.ds(0, 128)], slabs[slot],
                gsems[slot]).wait()

        def extract(slot, lbl, tstep, colbuf, col):
            lane = jnp.broadcast_to(lbl & 127, (16,))
            tlane = jnp.broadcast_to(tstep, (16,))
            colv = jnp.broadcast_to(col, (16,))
            for q in range(4):
                vals = plsc.load_gather(slabs[slot], [rows16[q], lane])
                w = plsc.load_gather(emb_v, [rhalf[q], tlane])
                lo = plsc.bitcast(lax.shift_left(w, 16), jnp.float32)
                hi = plsc.bitcast(w & himask, jnp.float32)
                emb = jnp.where(odd16[q], hi, lo)
                plsc.store_scatter(colbuf, [rows16[q], colv], vals + emb)

        out_copies = [None, None]
        for s in range(_NSEG):
            cb = s % 2
            if out_copies[cb] is not None:
                out_copies[cb].wait()
            lv0 = labels_v[pl.ds(s * _SEG, 16)]
            for j in range(_RING):
                fire(j, lv0[j])

            def group(c, carry, s=s, cb=cb):
                lv = labels_v[pl.ds(s * _SEG + c * _RING, 16)]
                tv = times_v[pl.ds(s * _SEG + c * _RING, 16)]
                for j in range(_RING):
                    wait_slot(j)
                    extract(j, lv[j], tv[j], cols[cb], c * _RING + j)

                    @pl.when(c < _NGRP - 1)
                    def _():
                        fire(j, lv[_RING + j])
                return carry

            lax.fori_loop(0, _NGRP, group, 0)
            out_copies[cb] = pltpu.async_copy(
                cols[cb],
                out_hbm.at[:, pl.ds(base + s * _SEG, _SEG)],
                osems[cb],
            )
        out_copies[0].wait()
        out_copies[1].wait()

    return _sc_gather


def kernel(timesteps, labels, class_embedding):
    embp = _emb_table()
    out_t = _make_sc_gather()(labels, timesteps, embp, class_embedding.T)
    return out_t.T
